# Initial kernel scaffold; baseline (speedup 1.0000x reference)
import jax
import jax.numpy as jnp
from jax.experimental import pallas as pl


def kernel(x, t_emb, router_w, router_t_w, router_bias, sw1, sw3, sw2, ew1, ew2):
    raise NotImplementedError("write your pallas kernel here")


# speedup vs baseline: 1.2721x; 1.2721x over previous
import functools

import jax
import jax.numpy as jnp
from jax.experimental import pallas as pl
from jax.experimental.pallas import tpu as pltpu

B, T, D = 1, 2048, 2048
E, TOPK = 8, 2
SH, RH = 4096, 1024
N = B * T
M = N * TOPK
BM = 256
M_PAD = M + E * BM
NB = M_PAD // BM


def _router_body(x_ref, rw_ref, rtw_ref, rb_ref, temb_ref, idx_ref, gates_ref):
    x = x_ref[...]
    logits = jax.lax.dot_general(
        x, rw_ref[...], (((1,), (1,)), ((), ())),
        preferred_element_type=jnp.float32)
    t_bias = jax.lax.dot_general(
        temb_ref[...], rtw_ref[...], (((1,), (1,)), ((), ())),
        preferred_element_type=jnp.float32)
    s = jax.nn.sigmoid(logits + t_bias)
    sel = s + rb_ref[...]
    lane = jax.lax.broadcasted_iota(jnp.int32, sel.shape, 1)
    i0 = jnp.argmax(sel, axis=1).astype(jnp.int32)
    sel2 = jnp.where(lane == i0[:, None], -jnp.inf, sel)
    i1 = jnp.argmax(sel2, axis=1).astype(jnp.int32)
    s0 = jnp.sum(jnp.where(lane == i0[:, None], s, 0.0), axis=1)
    s1 = jnp.sum(jnp.where(lane == i1[:, None], s, 0.0), axis=1)
    denom = s0 + s1
    safe = denom > 1e-9
    g0 = jnp.where(safe, s0 / (denom + 1e-9), 1.0 / TOPK)
    g1 = jnp.where(safe, s1 / (denom + 1e-9), 1.0 / TOPK)
    idx_ref[...] = jnp.stack([i0, i1], axis=1)
    gates_ref[...] = jnp.stack([g0, g1], axis=1)


def _router(x_flat, router_w, router_t_w, router_bias, t_emb):
    TB = 256
    return pl.pallas_call(
        _router_body,
        grid=(N // TB,),
        in_specs=[
            pl.BlockSpec((TB, D), lambda i: (i, 0)),
            pl.BlockSpec((E, D), lambda i: (0, 0)),
            pl.BlockSpec((E, D), lambda i: (0, 0)),
            pl.BlockSpec((1, E), lambda i: (0, 0)),
            pl.BlockSpec((1, D), lambda i: (0, 0)),
        ],
        out_specs=[
            pl.BlockSpec((TB, TOPK), lambda i: (i, 0)),
            pl.BlockSpec((TB, TOPK), lambda i: (i, 0)),
        ],
        out_shape=[
            jax.ShapeDtypeStruct((N, TOPK), jnp.int32),
            jax.ShapeDtypeStruct((N, TOPK), jnp.float32),
        ],
    )(x_flat, router_w, router_t_w, router_bias.reshape(1, E),
      t_emb.reshape(1, D))


def _dispatch_meta(idx):
    eflat = idx.reshape(M)
    oh = (eflat[:, None] == jnp.arange(E, dtype=jnp.int32)[None, :])
    ohi = oh.astype(jnp.int32)
    rank = jnp.sum((jnp.cumsum(ohi, axis=0) - ohi) * ohi, axis=1)
    counts = jnp.sum(ohi, axis=0)
    padc = ((counts + BM - 1) // BM) * BM
    pend = jnp.cumsum(padc)
    pstart = pend - padc
    p = pstart[eflat] + rank
    bstart = jnp.arange(NB, dtype=jnp.int32) * BM
    be_raw = jnp.searchsorted(pend, bstart, side='right').astype(jnp.int32)
    active = bstart < pend[-1]
    last_blk = jnp.maximum(pend[-1] // BM - 1, 0).astype(jnp.int32)
    last_e = jnp.clip(jnp.searchsorted(pend, last_blk * BM, side='right'),
                      0, E - 1).astype(jnp.int32)
    be = jnp.where(active, jnp.clip(be_raw, 0, E - 1), last_e)
    blk_x = jnp.where(active, jnp.arange(NB, dtype=jnp.int32), last_blk)
    return p, be, blk_x, active.astype(jnp.int32)


def _shared_up_body(x_ref, w1_ref, w3_ref, h_ref):
    x = x_ref[...].astype(jnp.bfloat16)
    w1 = w1_ref[...].astype(jnp.bfloat16)
    w3 = w3_ref[...].astype(jnp.bfloat16)
    h1 = jax.lax.dot_general(x, w1, (((1,), (1,)), ((), ())),
                             preferred_element_type=jnp.float32)
    h3 = jax.lax.dot_general(x, w3, (((1,), (1,)), ((), ())),
                             preferred_element_type=jnp.float32)
    h_ref[...] = (h1 * jax.nn.sigmoid(h1) * h3).astype(jnp.bfloat16)


def _shared_up(x_flat, sw1, sw3):
    TB, SB = 256, 1024
    return pl.pallas_call(
        _shared_up_body,
        grid=(SH // SB, N // TB),
        in_specs=[
            pl.BlockSpec((TB, D), lambda s, i: (i, 0)),
            pl.BlockSpec((SB, D), lambda s, i: (s, 0)),
            pl.BlockSpec((SB, D), lambda s, i: (s, 0)),
        ],
        out_specs=pl.BlockSpec((TB, SB), lambda s, i: (i, s)),
        out_shape=jax.ShapeDtypeStruct((N, SH), jnp.bfloat16),
    )(x_flat, sw1, sw3)


def _gmlp_body(be_ref, blk_ref, act_ref, xs_ref, w1_ref, w2_ref, y_ref):
    b = pl.program_id(0)

    @pl.when(act_ref[b] != 0)
    def _():
        x = xs_ref[...].astype(jnp.bfloat16)
        w1 = w1_ref[0].astype(jnp.bfloat16)
        h = jax.lax.dot_general(x, w1, (((1,), (1,)), ((), ())),
                                preferred_element_type=jnp.float32)
        h = (0.5 * h * (1.0 + jax.lax.erf(h * (2.0 ** -0.5)))).astype(jnp.bfloat16)
        w2 = w2_ref[0].astype(jnp.bfloat16)
        y = jax.lax.dot_general(h, w2, (((1,), (1,)), ((), ())),
                                preferred_element_type=jnp.float32)
        y_ref[...] = y


def _grouped_mlp(x_sorted, ew1, ew2, be, blk_x, active):
    grid_spec = pltpu.PrefetchScalarGridSpec(
        num_scalar_prefetch=3,
        grid=(NB,),
        in_specs=[
            pl.BlockSpec((BM, D), lambda b, be, blk, act: (blk[b], 0)),
            pl.BlockSpec((1, RH, D), lambda b, be, blk, act: (be[b], 0, 0)),
            pl.BlockSpec((1, D, RH), lambda b, be, blk, act: (be[b], 0, 0)),
        ],
        out_specs=pl.BlockSpec((BM, D), lambda b, be, blk, act: (b, 0)),
    )
    return pl.pallas_call(
        _gmlp_body,
        grid_spec=grid_spec,
        out_shape=jax.ShapeDtypeStruct((M_PAD, D), jnp.float32),
    )(be, blk_x, active, x_sorted, ew1, ew2)


def _combine_body(h_ref, w2_ref, yp_ref, g_ref, o_ref):
    k = pl.program_id(1)
    h = h_ref[...]
    w2 = w2_ref[...].astype(jnp.bfloat16)
    acc = jax.lax.dot_general(h, w2, (((1,), (1,)), ((), ())),
                              preferred_element_type=jnp.float32)

    @pl.when(k == 0)
    def _():
        yp = yp_ref[...]
        g = g_ref[...]
        y0 = yp[:, :D]
        y1 = yp[:, D:]
        routed = g[:, 0:1] * y0 + g[:, 1:2] * y1
        o_ref[...] = routed

    o_ref[...] += acc
    @pl.when(k == pl.num_programs(1) - 1)
    def _():
        o_ref[...] *= 1.0 / (1.0 + TOPK)


def _combine(h_sh, sw2, y_pair, gates):
    TB, SB = 256, 1024
    return pl.pallas_call(
        _combine_body,
        grid=(N // TB, SH // SB),
        in_specs=[
            pl.BlockSpec((TB, SB), lambda i, k: (i, k)),
            pl.BlockSpec((D, SB), lambda i, k: (0, k)),
            pl.BlockSpec((TB, 2 * D), lambda i, k: (i, 0)),
            pl.BlockSpec((TB, TOPK), lambda i, k: (i, 0)),
        ],
        out_specs=pl.BlockSpec((TB, D), lambda i, k: (i, 0)),
        out_shape=jax.ShapeDtypeStruct((N, D), jnp.float32),
    )(h_sh, sw2, y_pair, gates)


def kernel(x, t_emb, router_w, router_t_w, router_bias, sw1, sw3, sw2, ew1, ew2):
    x_flat = x.reshape(N, D)
    idx, gates = _router(x_flat, router_w, router_t_w, router_bias, t_emb)
    p, be, blk_x, active = _dispatch_meta(idx)

    tflat = jnp.arange(M, dtype=jnp.int32) // TOPK
    x_sorted = jnp.zeros((M_PAD, D), jnp.float32).at[p].set(x_flat[tflat])

    h_sh = _shared_up(x_flat, sw1, sw3)
    y_sorted = _grouped_mlp(x_sorted, ew1, ew2, be, blk_x, active)

    y_pair = y_sorted[p].reshape(N, 2 * D)

    out = _combine(h_sh, sw2, y_pair, gates)
    return out.reshape(B, T, D)


# speedup vs baseline: 1.5001x; 1.1792x over previous
import functools

import jax
import jax.numpy as jnp
from jax import lax
from jax.experimental import pallas as pl
from jax.experimental.pallas import tpu as pltpu
from jax.experimental.pallas import tpu_sc as plsc

B, T, D = 1, 2048, 2048
E, TOPK = 8, 2
SH, RH = 4096, 1024
N = B * T
M = N * TOPK
BM = 256
M_PAD = M + E * BM
NB = M_PAD // BM


def _router_body(x_ref, rw_ref, rtw_ref, rb_ref, temb_ref, idx_ref, gates_ref):
    x = x_ref[...]
    logits = jax.lax.dot_general(
        x, rw_ref[...], (((1,), (1,)), ((), ())),
        preferred_element_type=jnp.float32)
    t_bias = jax.lax.dot_general(
        temb_ref[...], rtw_ref[...], (((1,), (1,)), ((), ())),
        preferred_element_type=jnp.float32)
    s = jax.nn.sigmoid(logits + t_bias)
    sel = s + rb_ref[...]
    lane = jax.lax.broadcasted_iota(jnp.int32, sel.shape, 1)
    i0 = jnp.argmax(sel, axis=1).astype(jnp.int32)
    sel2 = jnp.where(lane == i0[:, None], -jnp.inf, sel)
    i1 = jnp.argmax(sel2, axis=1).astype(jnp.int32)
    s0 = jnp.sum(jnp.where(lane == i0[:, None], s, 0.0), axis=1)
    s1 = jnp.sum(jnp.where(lane == i1[:, None], s, 0.0), axis=1)
    denom = s0 + s1
    safe = denom > 1e-9
    g0 = jnp.where(safe, s0 / (denom + 1e-9), 1.0 / TOPK)
    g1 = jnp.where(safe, s1 / (denom + 1e-9), 1.0 / TOPK)
    idx_ref[...] = jnp.stack([i0, i1], axis=1)
    gates_ref[...] = jnp.stack([g0, g1], axis=1)


def _router(x_flat, router_w, router_t_w, router_bias, t_emb):
    TB = 256
    return pl.pallas_call(
        _router_body,
        grid=(N // TB,),
        in_specs=[
            pl.BlockSpec((TB, D), lambda i: (i, 0)),
            pl.BlockSpec((E, D), lambda i: (0, 0)),
            pl.BlockSpec((E, D), lambda i: (0, 0)),
            pl.BlockSpec((1, E), lambda i: (0, 0)),
            pl.BlockSpec((1, D), lambda i: (0, 0)),
        ],
        out_specs=[
            pl.BlockSpec((TB, TOPK), lambda i: (i, 0)),
            pl.BlockSpec((TB, TOPK), lambda i: (i, 0)),
        ],
        out_shape=[
            jax.ShapeDtypeStruct((N, TOPK), jnp.int32),
            jax.ShapeDtypeStruct((N, TOPK), jnp.float32),
        ],
    )(x_flat, router_w, router_t_w, router_bias.reshape(1, E),
      t_emb.reshape(1, D))


def _dispatch_meta(idx):
    eflat = idx.reshape(M)
    oh = (eflat[:, None] == jnp.arange(E, dtype=jnp.int32)[None, :])
    ohi = oh.astype(jnp.int32)
    rank = jnp.sum((jnp.cumsum(ohi, axis=0) - ohi) * ohi, axis=1)
    counts = jnp.sum(ohi, axis=0)
    padc = ((counts + BM - 1) // BM) * BM
    pend = jnp.cumsum(padc)
    pstart = pend - padc
    p = pstart[eflat] + rank
    bstart = jnp.arange(NB, dtype=jnp.int32) * BM
    be_raw = jnp.searchsorted(pend, bstart, side='right').astype(jnp.int32)
    active = bstart < pend[-1]
    last_blk = jnp.maximum(pend[-1] // BM - 1, 0).astype(jnp.int32)
    last_e = jnp.clip(jnp.searchsorted(pend, last_blk * BM, side='right'),
                      0, E - 1).astype(jnp.int32)
    be = jnp.where(active, jnp.clip(be_raw, 0, E - 1), last_e)
    blk_x = jnp.where(active, jnp.arange(NB, dtype=jnp.int32), last_blk)
    return p, be, blk_x, active.astype(jnp.int32)


_SC_MESH = plsc.VectorSubcoreMesh(core_axis_name="c", subcore_axis_name="s")
_NW = 32
_CH = 32


def _sc_dispatch(x_flat, p0, p1):
    tok_per_w = N // _NW
    n_ch = tok_per_w // _CH

    @functools.partial(
        pl.kernel, mesh=_SC_MESH,
        out_type=jax.ShapeDtypeStruct((M_PAD, D), jnp.float32),
        scratch_types=[
            pltpu.VMEM((_CH,), jnp.int32),
            pltpu.VMEM((_CH,), jnp.int32),
            pltpu.VMEM((_CH, D), jnp.float32),
            pltpu.SemaphoreType.DMA,
        ],
    )
    def k(x_hbm, p0_hbm, p1_hbm, out_hbm, i0_v, i1_v, rows_v, sem):
        wid = lax.axis_index("s") * 2 + lax.axis_index("c")
        for c in range(n_ch):
            base = wid * tok_per_w + c * _CH
            pltpu.sync_copy(x_hbm.at[pl.ds(base, _CH)], rows_v)
            pltpu.sync_copy(p0_hbm.at[pl.ds(base, _CH)], i0_v)
            pltpu.sync_copy(p1_hbm.at[pl.ds(base, _CH)], i1_v)
            cp0 = pltpu.async_copy(rows_v, out_hbm.at[i0_v], sem)
            cp1 = pltpu.async_copy(rows_v, out_hbm.at[i1_v], sem)
            cp0.wait()
            cp1.wait()

    return k(x_flat, p0, p1)


def _sc_combine_gather(y_sorted, p):
    rows_per_w = M // _NW
    n_ch = rows_per_w // _CH

    @functools.partial(
        pl.kernel, mesh=_SC_MESH,
        out_type=jax.ShapeDtypeStruct((M, D), jnp.float32),
        scratch_types=[
            pltpu.VMEM((_CH,), jnp.int32),
            pltpu.VMEM((_CH, D), jnp.float32),
            pltpu.SemaphoreType.DMA,
        ],
    )
    def k(y_hbm, p_hbm, out_hbm, idx_v, rows_v, sem):
        wid = lax.axis_index("s") * 2 + lax.axis_index("c")
        for c in range(n_ch):
            base = wid * rows_per_w + c * _CH
            pltpu.sync_copy(p_hbm.at[pl.ds(base, _CH)], idx_v)
            pltpu.async_copy(y_hbm.at[idx_v], rows_v, sem).wait()
            pltpu.sync_copy(rows_v, out_hbm.at[pl.ds(base, _CH)])

    return k(y_sorted, p)


def _shared_up_body(x_ref, w1_ref, w3_ref, h_ref):
    x = x_ref[...].astype(jnp.bfloat16)
    w1 = w1_ref[...].astype(jnp.bfloat16)
    w3 = w3_ref[...].astype(jnp.bfloat16)
    h1 = jax.lax.dot_general(x, w1, (((1,), (1,)), ((), ())),
                             preferred_element_type=jnp.float32)
    h3 = jax.lax.dot_general(x, w3, (((1,), (1,)), ((), ())),
                             preferred_element_type=jnp.float32)
    h_ref[...] = (h1 * jax.nn.sigmoid(h1) * h3).astype(jnp.bfloat16)


def _shared_up(x_flat, sw1, sw3):
    TB, SB = 256, 1024
    return pl.pallas_call(
        _shared_up_body,
        grid=(SH // SB, N // TB),
        in_specs=[
            pl.BlockSpec((TB, D), lambda s, i: (i, 0)),
            pl.BlockSpec((SB, D), lambda s, i: (s, 0)),
            pl.BlockSpec((SB, D), lambda s, i: (s, 0)),
        ],
        out_specs=pl.BlockSpec((TB, SB), lambda s, i: (i, s)),
        out_shape=jax.ShapeDtypeStruct((N, SH), jnp.bfloat16),
    )(x_flat, sw1, sw3)


def _gmlp_body(be_ref, blk_ref, act_ref, xs_ref, w1_ref, w2_ref, y_ref):
    b = pl.program_id(0)

    @pl.when(act_ref[b] != 0)
    def _():
        x = xs_ref[...].astype(jnp.bfloat16)
        w1 = w1_ref[0].astype(jnp.bfloat16)
        h = jax.lax.dot_general(x, w1, (((1,), (1,)), ((), ())),
                                preferred_element_type=jnp.float32)
        h = (0.5 * h * (1.0 + jax.lax.erf(h * (2.0 ** -0.5)))).astype(jnp.bfloat16)
        w2 = w2_ref[0].astype(jnp.bfloat16)
        y = jax.lax.dot_general(h, w2, (((1,), (1,)), ((), ())),
                                preferred_element_type=jnp.float32)
        y_ref[...] = y


def _grouped_mlp(x_sorted, ew1, ew2, be, blk_x, active):
    grid_spec = pltpu.PrefetchScalarGridSpec(
        num_scalar_prefetch=3,
        grid=(NB,),
        in_specs=[
            pl.BlockSpec((BM, D), lambda b, be, blk, act: (blk[b], 0)),
            pl.BlockSpec((1, RH, D), lambda b, be, blk, act: (be[b], 0, 0)),
            pl.BlockSpec((1, D, RH), lambda b, be, blk, act: (be[b], 0, 0)),
        ],
        out_specs=pl.BlockSpec((BM, D), lambda b, be, blk, act: (b, 0)),
    )
    return pl.pallas_call(
        _gmlp_body,
        grid_spec=grid_spec,
        out_shape=jax.ShapeDtypeStruct((M_PAD, D), jnp.float32),
    )(be, blk_x, active, x_sorted, ew1, ew2)


def _combine_body(h_ref, w2_ref, yp_ref, g_ref, o_ref):
    k = pl.program_id(1)
    h = h_ref[...]
    w2 = w2_ref[...].astype(jnp.bfloat16)
    acc = jax.lax.dot_general(h, w2, (((1,), (1,)), ((), ())),
                              preferred_element_type=jnp.float32)

    @pl.when(k == 0)
    def _():
        yp = yp_ref[...]
        g = g_ref[...]
        y0 = yp[:, :D]
        y1 = yp[:, D:]
        routed = g[:, 0:1] * y0 + g[:, 1:2] * y1
        o_ref[...] = routed

    o_ref[...] += acc
    @pl.when(k == pl.num_programs(1) - 1)
    def _():
        o_ref[...] *= 1.0 / (1.0 + TOPK)


def _combine(h_sh, sw2, y_pair, gates):
    TB, SB = 256, 1024
    return pl.pallas_call(
        _combine_body,
        grid=(N // TB, SH // SB),
        in_specs=[
            pl.BlockSpec((TB, SB), lambda i, k: (i, k)),
            pl.BlockSpec((D, SB), lambda i, k: (0, k)),
            pl.BlockSpec((TB, 2 * D), lambda i, k: (i, 0)),
            pl.BlockSpec((TB, TOPK), lambda i, k: (i, 0)),
        ],
        out_specs=pl.BlockSpec((TB, D), lambda i, k: (i, 0)),
        out_shape=jax.ShapeDtypeStruct((N, D), jnp.float32),
    )(h_sh, sw2, y_pair, gates)


def kernel(x, t_emb, router_w, router_t_w, router_bias, sw1, sw3, sw2, ew1, ew2):
    x_flat = x.reshape(N, D)
    idx, gates = _router(x_flat, router_w, router_t_w, router_bias, t_emb)
    p, be, blk_x, active = _dispatch_meta(idx)

    pr = p.reshape(N, TOPK)
    x_sorted = _sc_dispatch(x_flat, pr[:, 0], pr[:, 1])

    h_sh = _shared_up(x_flat, sw1, sw3)
    y_sorted = _grouped_mlp(x_sorted, ew1, ew2, be, blk_x, active)

    y_pair = _sc_combine_gather(y_sorted, p).reshape(N, 2 * D)

    out = _combine(h_sh, sw2, y_pair, gates)
    return out.reshape(B, T, D)


# speedup vs baseline: 1.7016x; 1.1343x over previous
import functools

import jax
import jax.numpy as jnp
from jax import lax
from jax.experimental import pallas as pl
from jax.experimental.pallas import tpu as pltpu
from jax.experimental.pallas import tpu_sc as plsc

B, T, D = 1, 2048, 2048
E, TOPK = 8, 2
SH, RH = 4096, 1024
N = B * T
M = N * TOPK
BM = 256
M_PAD = M + E * BM
NB = M_PAD // BM


def _router_body(x_ref, rw_ref, rtw_ref, rb_ref, temb_ref,
                 idx_ref, gates_ref, xbf_ref):
    x = x_ref[...]
    xbf_ref[...] = x.astype(jnp.bfloat16)
    logits = jax.lax.dot_general(
        x, rw_ref[...], (((1,), (1,)), ((), ())),
        preferred_element_type=jnp.float32)
    t_bias = jax.lax.dot_general(
        temb_ref[...], rtw_ref[...], (((1,), (1,)), ((), ())),
        preferred_element_type=jnp.float32)
    s = jax.nn.sigmoid(logits + t_bias)
    sel = s + rb_ref[...]
    lane = jax.lax.broadcasted_iota(jnp.int32, sel.shape, 1)
    i0 = jnp.argmax(sel, axis=1).astype(jnp.int32)
    sel2 = jnp.where(lane == i0[:, None], -jnp.inf, sel)
    i1 = jnp.argmax(sel2, axis=1).astype(jnp.int32)
    s0 = jnp.sum(jnp.where(lane == i0[:, None], s, 0.0), axis=1)
    s1 = jnp.sum(jnp.where(lane == i1[:, None], s, 0.0), axis=1)
    denom = s0 + s1
    safe = denom > 1e-9
    g0 = jnp.where(safe, s0 / (denom + 1e-9), 1.0 / TOPK)
    g1 = jnp.where(safe, s1 / (denom + 1e-9), 1.0 / TOPK)
    idx_ref[...] = jnp.stack([i0, i1], axis=1)
    gates_ref[...] = jnp.stack([g0, g1], axis=1)


def _router(x_flat, router_w, router_t_w, router_bias, t_emb):
    TB = 256
    return pl.pallas_call(
        _router_body,
        grid=(N // TB,),
        in_specs=[
            pl.BlockSpec((TB, D), lambda i: (i, 0)),
            pl.BlockSpec((E, D), lambda i: (0, 0)),
            pl.BlockSpec((E, D), lambda i: (0, 0)),
            pl.BlockSpec((1, E), lambda i: (0, 0)),
            pl.BlockSpec((1, D), lambda i: (0, 0)),
        ],
        out_specs=[
            pl.BlockSpec((TB, TOPK), lambda i: (i, 0)),
            pl.BlockSpec((TB, TOPK), lambda i: (i, 0)),
            pl.BlockSpec((TB, D), lambda i: (i, 0)),
        ],
        out_shape=[
            jax.ShapeDtypeStruct((N, TOPK), jnp.int32),
            jax.ShapeDtypeStruct((N, TOPK), jnp.float32),
            jax.ShapeDtypeStruct((N, D), jnp.bfloat16),
        ],
    )(x_flat, router_w, router_t_w, router_bias.reshape(1, E),
      t_emb.reshape(1, D))


def _dispatch_meta(idx):
    eflat = idx.reshape(M)
    oh = (eflat[:, None] == jnp.arange(E, dtype=jnp.int32)[None, :])
    ohi = oh.astype(jnp.int32)
    rank = jnp.sum((jnp.cumsum(ohi, axis=0) - ohi) * ohi, axis=1)
    counts = jnp.sum(ohi, axis=0)
    padc = ((counts + BM - 1) // BM) * BM
    pend = jnp.cumsum(padc)
    pstart = pend - padc
    p = pstart[eflat] + rank
    bstart = jnp.arange(NB, dtype=jnp.int32) * BM
    be_raw = jnp.sum((bstart[:, None] >= pend[None, :]).astype(jnp.int32),
                     axis=1)
    active = bstart < pend[-1]
    last_blk = jnp.maximum(pend[-1] // BM - 1, 0).astype(jnp.int32)
    last_e = jnp.clip(
        jnp.sum((last_blk * BM >= pend).astype(jnp.int32)), 0, E - 1
    ).astype(jnp.int32)
    be = jnp.where(active, jnp.clip(be_raw, 0, E - 1), last_e)
    blk_x = jnp.where(active, jnp.arange(NB, dtype=jnp.int32), last_blk)
    return p, be, blk_x, active.astype(jnp.int32)


_SC_MESH = plsc.VectorSubcoreMesh(core_axis_name="c", subcore_axis_name="s")
_NW = 32
_CH = 32
_SL = D // 128


def _sc_dispatch(x_flat, p0, p1):
    tok_per_w = N // _NW
    n_ch = tok_per_w // _CH

    @functools.partial(
        pl.kernel, mesh=_SC_MESH,
        out_type=jax.ShapeDtypeStruct((M_PAD, D), jnp.float32),
        scratch_types=[
            pltpu.VMEM((_CH,), jnp.int32),
            pltpu.VMEM((_CH,), jnp.int32),
            pltpu.VMEM((_CH, D), jnp.float32),
            pltpu.SemaphoreType.DMA,
        ],
    )
    def k(x_hbm, p0_hbm, p1_hbm, out_hbm, i0_v, i1_v, rows_v, sem):
        wid = lax.axis_index("s") * 2 + lax.axis_index("c")
        for c in range(n_ch):
            base = wid * tok_per_w + c * _CH
            pltpu.sync_copy(x_hbm.at[pl.ds(base, _CH)], rows_v)
            pltpu.sync_copy(p0_hbm.at[pl.ds(base, _CH)], i0_v)
            pltpu.sync_copy(p1_hbm.at[pl.ds(base, _CH)], i1_v)
            cp0 = pltpu.async_copy(rows_v, out_hbm.at[i0_v], sem)
            cp1 = pltpu.async_copy(rows_v, out_hbm.at[i1_v], sem)
            cp0.wait()
            cp1.wait()

    return k(x_flat, p0, p1)


def _sc_combine_gather(y_sorted, p2):
    rows_per_w = M // _NW
    n_ch = rows_per_w // _CH

    @functools.partial(
        pl.kernel, mesh=_SC_MESH,
        out_type=jax.ShapeDtypeStruct((M, D), jnp.float32),
        scratch_types=[
            pltpu.VMEM((_CH,), jnp.int32),
            pltpu.VMEM((_CH, D), jnp.float32),
            pltpu.SemaphoreType.DMA,
        ],
    )
    def k(y_hbm, p_hbm, out_hbm, idx_v, rows_v, sem):
        wid = lax.axis_index("s") * 2 + lax.axis_index("c")
        for c in range(n_ch):
            base = wid * rows_per_w + c * _CH
            pltpu.sync_copy(p_hbm.at[pl.ds(base, _CH)], idx_v)
            pltpu.async_copy(y_hbm.at[idx_v], rows_v, sem).wait()
            pltpu.sync_copy(rows_v, out_hbm.at[pl.ds(base, _CH)])

    return k(y_sorted, p2)


def _shared_up_body(x_ref, w1_ref, w3_ref, sw2_ref,
                    h_ref, sw2bf_ref, w1bf, w3bf):
    i = pl.program_id(1)

    @pl.when(i == 0)
    def _():
        w1bf[...] = w1_ref[...].astype(jnp.bfloat16)
        w3bf[...] = w3_ref[...].astype(jnp.bfloat16)
        sw2bf_ref[...] = sw2_ref[...].astype(jnp.bfloat16)

    x = x_ref[...]
    h1 = jax.lax.dot_general(x, w1bf[...], (((1,), (1,)), ((), ())),
                             preferred_element_type=jnp.float32)
    h3 = jax.lax.dot_general(x, w3bf[...], (((1,), (1,)), ((), ())),
                             preferred_element_type=jnp.float32)
    h_ref[...] = (h1 * jax.nn.sigmoid(h1) * h3).astype(jnp.bfloat16)


def _shared_up(x_bf, sw1, sw3, sw2):
    TB, SB = 256, 512
    return pl.pallas_call(
        _shared_up_body,
        grid=(SH // SB, N // TB),
        in_specs=[
            pl.BlockSpec((TB, D), lambda s, i: (i, 0)),
            pl.BlockSpec((SB, D), lambda s, i: (s, 0)),
            pl.BlockSpec((SB, D), lambda s, i: (s, 0)),
            pl.BlockSpec((D, SB), lambda s, i: (0, s)),
        ],
        out_specs=[
            pl.BlockSpec((TB, SB), lambda s, i: (i, s)),
            pl.BlockSpec((D, SB), lambda s, i: (0, s)),
        ],
        out_shape=[
            jax.ShapeDtypeStruct((N, SH), jnp.bfloat16),
            jax.ShapeDtypeStruct((D, SH), jnp.bfloat16),
        ],
        scratch_shapes=[
            pltpu.VMEM((SB, D), jnp.bfloat16),
            pltpu.VMEM((SB, D), jnp.bfloat16),
        ],
    )(x_bf, sw1, sw3, sw2)


def _gmlp_body(be_ref, blk_ref, act_ref, xs_ref, w1_ref, w2_ref, y_ref,
               w1bf, w2bf):
    b = pl.program_id(0)
    changed = jnp.logical_or(b == 0, be_ref[b] != be_ref[jnp.maximum(b - 1, 0)])

    @pl.when(jnp.logical_and(act_ref[b] != 0, changed))
    def _():
        w1bf[...] = w1_ref[0].astype(jnp.bfloat16)
        w2bf[...] = w2_ref[0].astype(jnp.bfloat16)

    @pl.when(act_ref[b] != 0)
    def _():
        x = xs_ref[...].astype(jnp.bfloat16)
        h = jax.lax.dot_general(x, w1bf[...], (((1,), (1,)), ((), ())),
                                preferred_element_type=jnp.float32)
        h = (0.5 * h * (1.0 + jax.lax.erf(h * (2.0 ** -0.5)))).astype(jnp.bfloat16)
        y = jax.lax.dot_general(h, w2bf[...], (((1,), (1,)), ((), ())),
                                preferred_element_type=jnp.float32)
        y_ref[...] = y


def _grouped_mlp(x_sorted, ew1, ew2, be, blk_x, active):
    grid_spec = pltpu.PrefetchScalarGridSpec(
        num_scalar_prefetch=3,
        grid=(NB,),
        in_specs=[
            pl.BlockSpec((BM, D), lambda b, be, blk, act: (blk[b], 0)),
            pl.BlockSpec((1, RH, D), lambda b, be, blk, act: (be[b], 0, 0)),
            pl.BlockSpec((1, D, RH), lambda b, be, blk, act: (be[b], 0, 0)),
        ],
        out_specs=pl.BlockSpec((BM, D), lambda b, be, blk, act: (b, 0)),
        scratch_shapes=[
            pltpu.VMEM((RH, D), jnp.bfloat16),
            pltpu.VMEM((D, RH), jnp.bfloat16),
        ],
    )
    return pl.pallas_call(
        _gmlp_body,
        grid_spec=grid_spec,
        out_shape=jax.ShapeDtypeStruct((M_PAD, D), jnp.float32),
    )(be, blk_x, active, x_sorted, ew1, ew2)


def _combine_body(h_ref, w2_ref, y0_ref, y1_ref, g_ref, o_ref):
    h = h_ref[...]
    acc = jax.lax.dot_general(h, w2_ref[...], (((1,), (1,)), ((), ())),
                              preferred_element_type=jnp.float32)
    g = g_ref[...]
    y0 = y0_ref[0]
    y1 = y1_ref[0]
    o_ref[...] = (acc + g[:, 0:1] * y0 + g[:, 1:2] * y1) * (1.0 / (1.0 + TOPK))


def _combine(h_sh, sw2bf, y_slots, gates):
    TB = 256
    return pl.pallas_call(
        _combine_body,
        grid=(N // TB,),
        in_specs=[
            pl.BlockSpec((TB, SH), lambda i: (i, 0)),
            pl.BlockSpec((D, SH), lambda i: (0, 0)),
            pl.BlockSpec((1, TB, D), lambda i: (0, i, 0)),
            pl.BlockSpec((1, TB, D), lambda i: (1, i, 0)),
            pl.BlockSpec((TB, TOPK), lambda i: (i, 0)),
        ],
        out_specs=pl.BlockSpec((TB, D), lambda i: (i, 0)),
        out_shape=jax.ShapeDtypeStruct((N, D), jnp.float32),
    )(h_sh, sw2bf, y_slots, y_slots, gates)


def kernel(x, t_emb, router_w, router_t_w, router_bias, sw1, sw3, sw2, ew1, ew2):
    x_flat = x.reshape(N, D)
    idx, gates, x_bf = _router(x_flat, router_w, router_t_w, router_bias, t_emb)
    p, be, blk_x, active = _dispatch_meta(idx)

    pr = p.reshape(N, TOPK)
    x_sorted = _sc_dispatch(x_flat, pr[:, 0], pr[:, 1])

    h_sh, sw2bf = _shared_up(x_bf, sw1, sw3, sw2)
    y_sorted = _grouped_mlp(x_sorted, ew1, ew2, be, blk_x, active)

    p2 = jnp.concatenate([pr[:, 0], pr[:, 1]])
    y_slots = _sc_combine_gather(y_sorted, p2)

    out = _combine(h_sh, sw2bf, y_slots.reshape(TOPK, N, D), gates)
    return out.reshape(B, T, D)


# speedup vs baseline: 1.7886x; 1.0512x over previous
import functools

import jax
import jax.numpy as jnp
from jax import lax
from jax.experimental import pallas as pl
from jax.experimental.pallas import tpu as pltpu
from jax.experimental.pallas import tpu_sc as plsc

B, T, D = 1, 2048, 2048
E, TOPK = 8, 2
SH, RH = 4096, 1024
N = B * T
M = N * TOPK
BM = 256
M_PAD = M + E * BM
NB = M_PAD // BM


def _router_body(x_ref, rw_ref, rtw_ref, rb_ref, temb_ref,
                 idx_ref, gates_ref, xbf_ref):
    x = x_ref[...]
    xbf_ref[...] = x.astype(jnp.bfloat16)
    logits = jax.lax.dot_general(
        x, rw_ref[...], (((1,), (1,)), ((), ())),
        preferred_element_type=jnp.float32)
    t_bias = jax.lax.dot_general(
        temb_ref[...], rtw_ref[...], (((1,), (1,)), ((), ())),
        preferred_element_type=jnp.float32)
    s = jax.nn.sigmoid(logits + t_bias)
    sel = s + rb_ref[...]
    lane = jax.lax.broadcasted_iota(jnp.int32, sel.shape, 1)
    i0 = jnp.argmax(sel, axis=1).astype(jnp.int32)
    sel2 = jnp.where(lane == i0[:, None], -jnp.inf, sel)
    i1 = jnp.argmax(sel2, axis=1).astype(jnp.int32)
    s0 = jnp.sum(jnp.where(lane == i0[:, None], s, 0.0), axis=1)
    s1 = jnp.sum(jnp.where(lane == i1[:, None], s, 0.0), axis=1)
    denom = s0 + s1
    safe = denom > 1e-9
    g0 = jnp.where(safe, s0 / (denom + 1e-9), 1.0 / TOPK)
    g1 = jnp.where(safe, s1 / (denom + 1e-9), 1.0 / TOPK)
    idx_ref[...] = jnp.stack([i0, i1], axis=1)
    gates_ref[...] = jnp.stack([g0, g1], axis=1)


def _router(x_flat, router_w, router_t_w, router_bias, t_emb):
    TB = 256
    return pl.pallas_call(
        _router_body,
        grid=(N // TB,),
        in_specs=[
            pl.BlockSpec((TB, D), lambda i: (i, 0)),
            pl.BlockSpec((E, D), lambda i: (0, 0)),
            pl.BlockSpec((E, D), lambda i: (0, 0)),
            pl.BlockSpec((1, E), lambda i: (0, 0)),
            pl.BlockSpec((1, D), lambda i: (0, 0)),
        ],
        out_specs=[
            pl.BlockSpec((TB, TOPK), lambda i: (i, 0)),
            pl.BlockSpec((TB, TOPK), lambda i: (i, 0)),
            pl.BlockSpec((TB, D), lambda i: (i, 0)),
        ],
        out_shape=[
            jax.ShapeDtypeStruct((N, TOPK), jnp.int32),
            jax.ShapeDtypeStruct((N, TOPK), jnp.float32),
            jax.ShapeDtypeStruct((N, D), jnp.bfloat16),
        ],
    )(x_flat, router_w, router_t_w, router_bias.reshape(1, E),
      t_emb.reshape(1, D))


def _dispatch_meta(idx):
    eflat = idx.reshape(M)
    oh = (eflat[:, None] == jnp.arange(E, dtype=jnp.int32)[None, :])
    ohi = oh.astype(jnp.int32)
    rank = jnp.sum((jnp.cumsum(ohi, axis=0) - ohi) * ohi, axis=1)
    counts = jnp.sum(ohi, axis=0)
    padc = ((counts + BM - 1) // BM) * BM
    pend = jnp.cumsum(padc)
    pstart = pend - padc
    p = pstart[eflat] + rank
    bstart = jnp.arange(NB, dtype=jnp.int32) * BM
    be_raw = jnp.sum((bstart[:, None] >= pend[None, :]).astype(jnp.int32),
                     axis=1)
    active = bstart < pend[-1]
    last_blk = jnp.maximum(pend[-1] // BM - 1, 0).astype(jnp.int32)
    last_e = jnp.clip(
        jnp.sum((last_blk * BM >= pend).astype(jnp.int32)), 0, E - 1
    ).astype(jnp.int32)
    be = jnp.where(active, jnp.clip(be_raw, 0, E - 1), last_e)
    blk_x = jnp.where(active, jnp.arange(NB, dtype=jnp.int32), last_blk)
    return p, be, blk_x, active.astype(jnp.int32)


_SC_MESH = plsc.VectorSubcoreMesh(core_axis_name="c", subcore_axis_name="s")
_NW = 32
_CH = 32
_SL = D // 128


def _sc_dispatch(x_flat, p0, p1):
    tok_per_w = N // _NW
    n_ch = tok_per_w // _CH

    @functools.partial(
        pl.kernel, mesh=_SC_MESH,
        out_type=jax.ShapeDtypeStruct((M_PAD, D), jnp.float32),
        scratch_types=[
            pltpu.VMEM((_CH,), jnp.int32),
            pltpu.VMEM((_CH,), jnp.int32),
            pltpu.VMEM((_CH, D), jnp.float32),
            pltpu.SemaphoreType.DMA,
        ],
    )
    def k(x_hbm, p0_hbm, p1_hbm, out_hbm, i0_v, i1_v, rows_v, sem):
        wid = lax.axis_index("s") * 2 + lax.axis_index("c")
        for c in range(n_ch):
            base = wid * tok_per_w + c * _CH
            pltpu.sync_copy(x_hbm.at[pl.ds(base, _CH)], rows_v)
            pltpu.sync_copy(p0_hbm.at[pl.ds(base, _CH)], i0_v)
            pltpu.sync_copy(p1_hbm.at[pl.ds(base, _CH)], i1_v)
            cp0 = pltpu.async_copy(rows_v, out_hbm.at[i0_v], sem)
            cp1 = pltpu.async_copy(rows_v, out_hbm.at[i1_v], sem)
            cp0.wait()
            cp1.wait()

    return k(x_flat, p0, p1)


def _sc_combine_gather(y_sorted, p2):
    rows_per_w = M // _NW
    n_ch = rows_per_w // _CH

    @functools.partial(
        pl.kernel, mesh=_SC_MESH,
        out_type=jax.ShapeDtypeStruct((M, D), jnp.float32),
        scratch_types=[
            pltpu.VMEM((_CH,), jnp.int32),
            pltpu.VMEM((_CH, D), jnp.float32),
            pltpu.SemaphoreType.DMA,
        ],
    )
    def k(y_hbm, p_hbm, out_hbm, idx_v, rows_v, sem):
        wid = lax.axis_index("s") * 2 + lax.axis_index("c")
        for c in range(n_ch):
            base = wid * rows_per_w + c * _CH
            pltpu.sync_copy(p_hbm.at[pl.ds(base, _CH)], idx_v)
            pltpu.async_copy(y_hbm.at[idx_v], rows_v, sem).wait()
            pltpu.sync_copy(rows_v, out_hbm.at[pl.ds(base, _CH)])

    return k(y_sorted, p2)


def _shared_up_body(x_ref, w1_ref, w3_ref, sw2_ref,
                    h_ref, sw2bf_ref, w1bf, w3bf):
    i = pl.program_id(1)

    @pl.when(i == 0)
    def _():
        w1bf[...] = w1_ref[...].astype(jnp.bfloat16)
        w3bf[...] = w3_ref[...].astype(jnp.bfloat16)
        sw2bf_ref[...] = sw2_ref[...].astype(jnp.bfloat16)

    x = x_ref[...]
    h1 = jax.lax.dot_general(x, w1bf[...], (((1,), (1,)), ((), ())),
                             preferred_element_type=jnp.float32)
    h3 = jax.lax.dot_general(x, w3bf[...], (((1,), (1,)), ((), ())),
                             preferred_element_type=jnp.float32)
    h_ref[...] = (h1 * jax.nn.sigmoid(h1) * h3).astype(jnp.bfloat16)


def _shared_up(x_bf, sw1, sw3, sw2):
    TB, SB = 512, 512
    return pl.pallas_call(
        _shared_up_body,
        grid=(SH // SB, N // TB),
        in_specs=[
            pl.BlockSpec((TB, D), lambda s, i: (i, 0)),
            pl.BlockSpec((SB, D), lambda s, i: (s, 0)),
            pl.BlockSpec((SB, D), lambda s, i: (s, 0)),
            pl.BlockSpec((D, SB), lambda s, i: (0, s)),
        ],
        out_specs=[
            pl.BlockSpec((TB, SB), lambda s, i: (i, s)),
            pl.BlockSpec((D, SB), lambda s, i: (0, s)),
        ],
        out_shape=[
            jax.ShapeDtypeStruct((N, SH), jnp.bfloat16),
            jax.ShapeDtypeStruct((D, SH), jnp.bfloat16),
        ],
        scratch_shapes=[
            pltpu.VMEM((SB, D), jnp.bfloat16),
            pltpu.VMEM((SB, D), jnp.bfloat16),
        ],
    )(x_bf, sw1, sw3, sw2)


def _gmlp_body(be_ref, blk_ref, act_ref, xs_ref, w1_ref, w2_ref, y_ref,
               w1bf, w2bf):
    b = pl.program_id(0)
    changed = jnp.logical_or(b == 0, be_ref[b] != be_ref[jnp.maximum(b - 1, 0)])

    @pl.when(jnp.logical_and(act_ref[b] != 0, changed))
    def _():
        w1bf[...] = w1_ref[0].astype(jnp.bfloat16)
        w2bf[...] = w2_ref[0].astype(jnp.bfloat16)

    @pl.when(act_ref[b] != 0)
    def _():
        x = xs_ref[...].astype(jnp.bfloat16)
        h = jax.lax.dot_general(x, w1bf[...], (((1,), (1,)), ((), ())),
                                preferred_element_type=jnp.float32)
        h = (0.5 * h * (1.0 + jax.lax.erf(h * (2.0 ** -0.5)))).astype(jnp.bfloat16)
        y = jax.lax.dot_general(h, w2bf[...], (((1,), (1,)), ((), ())),
                                preferred_element_type=jnp.float32)
        y_ref[...] = y


def _grouped_mlp(x_sorted, ew1, ew2, be, blk_x, active):
    grid_spec = pltpu.PrefetchScalarGridSpec(
        num_scalar_prefetch=3,
        grid=(NB,),
        in_specs=[
            pl.BlockSpec((BM, D), lambda b, be, blk, act: (blk[b], 0)),
            pl.BlockSpec((1, RH, D), lambda b, be, blk, act: (be[b], 0, 0)),
            pl.BlockSpec((1, D, RH), lambda b, be, blk, act: (be[b], 0, 0)),
        ],
        out_specs=pl.BlockSpec((BM, D), lambda b, be, blk, act: (b, 0)),
        scratch_shapes=[
            pltpu.VMEM((RH, D), jnp.bfloat16),
            pltpu.VMEM((D, RH), jnp.bfloat16),
        ],
    )
    return pl.pallas_call(
        _gmlp_body,
        grid_spec=grid_spec,
        out_shape=jax.ShapeDtypeStruct((M_PAD, D), jnp.float32),
    )(be, blk_x, active, x_sorted, ew1, ew2)


def _combine_body(h_ref, w2_ref, y0_ref, y1_ref, g_ref, o_ref):
    h = h_ref[...]
    acc = jax.lax.dot_general(h, w2_ref[...], (((1,), (1,)), ((), ())),
                              preferred_element_type=jnp.float32)
    g = g_ref[...]
    y0 = y0_ref[0]
    y1 = y1_ref[0]
    o_ref[...] = (acc + g[:, 0:1] * y0 + g[:, 1:2] * y1) * (1.0 / (1.0 + TOPK))


def _combine(h_sh, sw2bf, y_slots, gates):
    TB = 256
    return pl.pallas_call(
        _combine_body,
        grid=(N // TB,),
        in_specs=[
            pl.BlockSpec((TB, SH), lambda i: (i, 0)),
            pl.BlockSpec((D, SH), lambda i: (0, 0)),
            pl.BlockSpec((1, TB, D), lambda i: (0, i, 0)),
            pl.BlockSpec((1, TB, D), lambda i: (1, i, 0)),
            pl.BlockSpec((TB, TOPK), lambda i: (i, 0)),
        ],
        out_specs=pl.BlockSpec((TB, D), lambda i: (i, 0)),
        out_shape=jax.ShapeDtypeStruct((N, D), jnp.float32),
    )(h_sh, sw2bf, y_slots, y_slots, gates)


def kernel(x, t_emb, router_w, router_t_w, router_bias, sw1, sw3, sw2, ew1, ew2):
    x_flat = x.reshape(N, D)
    idx, gates, x_bf = _router(x_flat, router_w, router_t_w, router_bias, t_emb)
    p, be, blk_x, active = _dispatch_meta(idx)

    pr = p.reshape(N, TOPK)
    x_sorted = _sc_dispatch(x_flat, pr[:, 0], pr[:, 1])

    h_sh, sw2bf = _shared_up(x_bf, sw1, sw3, sw2)
    y_sorted = _grouped_mlp(x_sorted, ew1, ew2, be, blk_x, active)

    p2 = jnp.concatenate([pr[:, 0], pr[:, 1]])
    y_slots = _sc_combine_gather(y_sorted, p2)

    out = _combine(h_sh, sw2bf, y_slots.reshape(TOPK, N, D), gates)
    return out.reshape(B, T, D)


# speedup vs baseline: 1.8940x; 1.0589x over previous
import functools

import jax
import jax.numpy as jnp
from jax import lax
from jax.experimental import pallas as pl
from jax.experimental.pallas import tpu as pltpu
from jax.experimental.pallas import tpu_sc as plsc

B, T, D = 1, 2048, 2048
E, TOPK = 8, 2
SH, RH = 4096, 1024
N = B * T
M = N * TOPK
BM = 256
M_PAD = M + E * BM
NB = M_PAD // BM


def _router_body(x_ref, rw_ref, rtw_ref, rb_ref, temb_ref,
                 gates_ref, xbf_ref, pall_ref, meta_ref, idxs):
    i = pl.program_id(0)
    nblk = pl.num_programs(0)
    x = x_ref[...]
    xbf_ref[...] = x.astype(jnp.bfloat16)
    logits = jax.lax.dot_general(
        rw_ref[...], x, (((1,), (1,)), ((), ())),
        preferred_element_type=jnp.float32)
    t_bias = jax.lax.dot_general(
        rtw_ref[...], temb_ref[...], (((1,), (1,)), ((), ())),
        preferred_element_type=jnp.float32)
    s = jax.nn.sigmoid(logits + t_bias)
    sel = s + rb_ref[...]
    erow = jax.lax.broadcasted_iota(jnp.int32, sel.shape, 0)
    i0 = jnp.argmax(sel, axis=0).astype(jnp.int32)
    sel2 = jnp.where(erow == i0[None, :], -jnp.inf, sel)
    i1 = jnp.argmax(sel2, axis=0).astype(jnp.int32)
    s0 = jnp.sum(jnp.where(erow == i0[None, :], s, 0.0), axis=0)
    s1 = jnp.sum(jnp.where(erow == i1[None, :], s, 0.0), axis=0)
    denom = s0 + s1
    safe = denom > 1e-9
    g0 = jnp.where(safe, s0 / (denom + 1e-9), 1.0 / TOPK)
    g1 = jnp.where(safe, s1 / (denom + 1e-9), 1.0 / TOPK)
    gates_ref[...] = jnp.stack([g0, g1], axis=1)
    tb = x.shape[0]
    idxs[0, pl.ds(i * tb, tb)] = i0
    idxs[1, pl.ds(i * tb, tb)] = i1

    @pl.when(i == nblk - 1)
    def _():
        e0 = idxs[0:1, :]
        e1 = idxs[1:2, :]
        eio = jax.lax.broadcasted_iota(jnp.int32, (E, N), 0)
        oh0 = (e0 == eio).astype(jnp.int32)
        oh1 = (e1 == eio).astype(jnp.int32)
        ohsum = oh0 + oh1
        c = ohsum
        sh = 1
        while sh < N:
            z = jnp.zeros((E, sh), jnp.int32)
            c = c + jnp.concatenate([z, c[:, :N - sh]], axis=1)
            sh *= 2
        cexc = c - ohsum
        counts = c[:, N - 1:N]
        padc = ((counts + BM - 1) // BM) * BM
        pe = padc
        sh = 1
        while sh < E:
            z2 = jnp.zeros((sh, 1), jnp.int32)
            pe = pe + jnp.concatenate([z2, pe[:E - sh, :]], axis=0)
            sh *= 2
        pstart = pe - padc
        base = pstart + cexc
        p0 = jnp.sum(oh0 * base, axis=0)
        p1 = jnp.sum(oh1 * base, axis=0)
        pall_ref[pl.ds(0, N)] = p0
        pall_ref[pl.ds(N, N)] = p1
        bstart = jax.lax.broadcasted_iota(jnp.int32, (1, NB), 1) * BM
        be_raw = jnp.sum((bstart >= pe).astype(jnp.int32), axis=0,
                         keepdims=True)
        pend_last = pe[E - 1:E, :]
        activeb = (bstart < pend_last).astype(jnp.int32)
        last_blk = jnp.maximum(pend_last // BM - 1, 0)
        last_e = jnp.clip(
            jnp.sum((last_blk * BM >= pe).astype(jnp.int32), axis=0,
                    keepdims=True), 0, E - 1)
        meta_ref[0:1, :] = jnp.where(activeb == 1,
                                     jnp.clip(be_raw, 0, E - 1), last_e)
        meta_ref[1:2, :] = jnp.where(
            activeb == 1, jax.lax.broadcasted_iota(jnp.int32, (1, NB), 1),
            last_blk)
        meta_ref[2:3, :] = activeb


def _router(x_flat, router_w, router_t_w, router_bias, t_emb):
    TB = 256
    return pl.pallas_call(
        _router_body,
        grid=(N // TB,),
        in_specs=[
            pl.BlockSpec((TB, D), lambda i: (i, 0)),
            pl.BlockSpec((E, D), lambda i: (0, 0)),
            pl.BlockSpec((E, D), lambda i: (0, 0)),
            pl.BlockSpec((E, 1), lambda i: (0, 0)),
            pl.BlockSpec((1, D), lambda i: (0, 0)),
        ],
        out_specs=[
            pl.BlockSpec((TB, TOPK), lambda i: (i, 0)),
            pl.BlockSpec((TB, D), lambda i: (i, 0)),
            pl.BlockSpec((2 * N,), lambda i: (0,)),
            pl.BlockSpec((3, NB), lambda i: (0, 0)),
        ],
        out_shape=[
            jax.ShapeDtypeStruct((N, TOPK), jnp.float32),
            jax.ShapeDtypeStruct((N, D), jnp.bfloat16),
            jax.ShapeDtypeStruct((2 * N,), jnp.int32),
            jax.ShapeDtypeStruct((3, NB), jnp.int32),
        ],
        scratch_shapes=[pltpu.VMEM((2, N), jnp.int32)],
    )(x_flat, router_w, router_t_w, router_bias.reshape(E, 1),
      t_emb.reshape(1, D))


def _sc_mesh():
    return plsc.VectorSubcoreMesh(core_axis_name="c", subcore_axis_name="s")

_NW = 32
_CH = 32
_SL = D // 128


def _sc_dispatch(x_flat, pall):
    tok_per_w = N // _NW
    n_ch = tok_per_w // _CH

    @functools.partial(
        pl.kernel, mesh=_sc_mesh(),
        out_type=jax.ShapeDtypeStruct((M_PAD, D), jnp.float32),
        scratch_types=[
            pltpu.VMEM((_CH,), jnp.int32),
            pltpu.VMEM((_CH,), jnp.int32),
            pltpu.VMEM((_CH, D), jnp.float32),
            pltpu.SemaphoreType.DMA,
        ],
    )
    def k(x_hbm, pall_hbm, out_hbm, i0_v, i1_v, rows_v, sem):
        wid = lax.axis_index("s") * 2 + lax.axis_index("c")
        for c in range(n_ch):
            base = wid * tok_per_w + c * _CH
            pltpu.sync_copy(x_hbm.at[pl.ds(base, _CH)], rows_v)
            pltpu.sync_copy(pall_hbm.at[pl.ds(base, _CH)], i0_v)
            pltpu.sync_copy(pall_hbm.at[pl.ds(N + base, _CH)], i1_v)
            cp0 = pltpu.async_copy(rows_v, out_hbm.at[i0_v], sem)
            cp1 = pltpu.async_copy(rows_v, out_hbm.at[i1_v], sem)
            cp0.wait()
            cp1.wait()

    return k(x_flat, pall)


def _sc_combine_gather(y_sorted, p2):
    rows_per_w = M // _NW
    n_ch = rows_per_w // _CH

    @functools.partial(
        pl.kernel, mesh=_sc_mesh(),
        out_type=jax.ShapeDtypeStruct((M, D), jnp.float32),
        scratch_types=[
            pltpu.VMEM((_CH,), jnp.int32),
            pltpu.VMEM((_CH, D), jnp.float32),
            pltpu.SemaphoreType.DMA,
        ],
    )
    def k(y_hbm, p_hbm, out_hbm, idx_v, rows_v, sem):
        wid = lax.axis_index("s") * 2 + lax.axis_index("c")
        for c in range(n_ch):
            base = wid * rows_per_w + c * _CH
            pltpu.sync_copy(p_hbm.at[pl.ds(base, _CH)], idx_v)
            pltpu.async_copy(y_hbm.at[idx_v], rows_v, sem).wait()
            pltpu.sync_copy(rows_v, out_hbm.at[pl.ds(base, _CH)])

    return k(y_sorted, p2)


def _shared_up_body(x_ref, w1_ref, w3_ref, sw2_ref,
                    h_ref, sw2bf_ref, w1bf, w3bf):
    i = pl.program_id(1)

    @pl.when(i == 0)
    def _():
        w1bf[...] = w1_ref[...].astype(jnp.bfloat16)
        w3bf[...] = w3_ref[...].astype(jnp.bfloat16)
        sw2bf_ref[...] = sw2_ref[...].astype(jnp.bfloat16)

    x = x_ref[...]
    h1 = jax.lax.dot_general(x, w1bf[...], (((1,), (1,)), ((), ())),
                             preferred_element_type=jnp.float32)
    h3 = jax.lax.dot_general(x, w3bf[...], (((1,), (1,)), ((), ())),
                             preferred_element_type=jnp.float32)
    h_ref[...] = (h1 * jax.nn.sigmoid(h1) * h3).astype(jnp.bfloat16)


def _shared_up(x_bf, sw1, sw3, sw2):
    TB, SB = 512, 512
    return pl.pallas_call(
        _shared_up_body,
        grid=(SH // SB, N // TB),
        in_specs=[
            pl.BlockSpec((TB, D), lambda s, i: (i, 0)),
            pl.BlockSpec((SB, D), lambda s, i: (s, 0)),
            pl.BlockSpec((SB, D), lambda s, i: (s, 0)),
            pl.BlockSpec((D, SB), lambda s, i: (0, s)),
        ],
        out_specs=[
            pl.BlockSpec((TB, SB), lambda s, i: (i, s)),
            pl.BlockSpec((D, SB), lambda s, i: (0, s)),
        ],
        out_shape=[
            jax.ShapeDtypeStruct((N, SH), jnp.bfloat16),
            jax.ShapeDtypeStruct((D, SH), jnp.bfloat16),
        ],
        scratch_shapes=[
            pltpu.VMEM((SB, D), jnp.bfloat16),
            pltpu.VMEM((SB, D), jnp.bfloat16),
        ],
    )(x_bf, sw1, sw3, sw2)


def _gmlp_body(meta_ref, xs_ref, w1_ref, w2_ref, y_ref, w1bf, w2bf):
    b = pl.program_id(0)
    changed = jnp.logical_or(
        b == 0, meta_ref[0, b] != meta_ref[0, jnp.maximum(b - 1, 0)])

    @pl.when(jnp.logical_and(meta_ref[2, b] != 0, changed))
    def _():
        w1bf[...] = w1_ref[0].astype(jnp.bfloat16)
        w2bf[...] = w2_ref[0].astype(jnp.bfloat16)

    @pl.when(meta_ref[2, b] != 0)
    def _():
        x = xs_ref[...].astype(jnp.bfloat16)
        h = jax.lax.dot_general(x, w1bf[...], (((1,), (1,)), ((), ())),
                                preferred_element_type=jnp.float32)
        h = (0.5 * h * (1.0 + jax.lax.erf(h * (2.0 ** -0.5)))).astype(jnp.bfloat16)
        y = jax.lax.dot_general(h, w2bf[...], (((1,), (1,)), ((), ())),
                                preferred_element_type=jnp.float32)
        y_ref[...] = y


def _grouped_mlp(x_sorted, ew1, ew2, meta):
    grid_spec = pltpu.PrefetchScalarGridSpec(
        num_scalar_prefetch=1,
        grid=(NB,),
        in_specs=[
            pl.BlockSpec((BM, D), lambda b, meta: (meta[1, b], 0)),
            pl.BlockSpec((1, RH, D), lambda b, meta: (meta[0, b], 0, 0)),
            pl.BlockSpec((1, D, RH), lambda b, meta: (meta[0, b], 0, 0)),
        ],
        out_specs=pl.BlockSpec((BM, D), lambda b, meta: (b, 0)),
        scratch_shapes=[
            pltpu.VMEM((RH, D), jnp.bfloat16),
            pltpu.VMEM((D, RH), jnp.bfloat16),
        ],
    )
    return pl.pallas_call(
        _gmlp_body,
        grid_spec=grid_spec,
        out_shape=jax.ShapeDtypeStruct((M_PAD, D), jnp.float32),
    )(meta, x_sorted, ew1, ew2)


def _combine_body(h_ref, w2_ref, y0_ref, y1_ref, g_ref, o_ref):
    h = h_ref[...]
    acc = jax.lax.dot_general(h, w2_ref[...], (((1,), (1,)), ((), ())),
                              preferred_element_type=jnp.float32)
    g = g_ref[...]
    y0 = y0_ref[0]
    y1 = y1_ref[0]
    o_ref[...] = (acc + g[:, 0:1] * y0 + g[:, 1:2] * y1) * (1.0 / (1.0 + TOPK))


def _combine(h_sh, sw2bf, y_slots, gates):
    TB = 256
    return pl.pallas_call(
        _combine_body,
        grid=(N // TB,),
        in_specs=[
            pl.BlockSpec((TB, SH), lambda i: (i, 0)),
            pl.BlockSpec((D, SH), lambda i: (0, 0)),
            pl.BlockSpec((1, TB, D), lambda i: (0, i, 0)),
            pl.BlockSpec((1, TB, D), lambda i: (1, i, 0)),
            pl.BlockSpec((TB, TOPK), lambda i: (i, 0)),
        ],
        out_specs=pl.BlockSpec((TB, D), lambda i: (i, 0)),
        out_shape=jax.ShapeDtypeStruct((N, D), jnp.float32),
    )(h_sh, sw2bf, y_slots, y_slots, gates)


def kernel(x, t_emb, router_w, router_t_w, router_bias, sw1, sw3, sw2, ew1, ew2):
    x_flat = x.reshape(N, D)
    gates, x_bf, pall, meta = _router(x_flat, router_w, router_t_w,
                                      router_bias, t_emb)

    x_sorted = _sc_dispatch(x_flat, pall)
    h_sh, sw2bf = _shared_up(x_bf, sw1, sw3, sw2)
    y_sorted = _grouped_mlp(x_sorted, ew1, ew2, meta)

    y_slots = _sc_combine_gather(y_sorted, pall)

    out = _combine(h_sh, sw2bf, y_slots.reshape(TOPK, N, D), gates)
    return out.reshape(B, T, D)
